# baseline (speedup 1.0000x reference)
import jax
import jax.numpy as jnp
from jax import lax
from jax.experimental import pallas as pl
from jax.experimental.pallas import tpu as pltpu
from jax.experimental.pallas import tpu_sc as plsc

_NW = 32
_BATCH = 16384
_DIM = 64
_NROW = 1000000
_B_PER_W = _BATCH // _NW
_NBUF = 6
_LOOK = 4
_TAIL_SLOT = _NBUF
_TAIL_BLOCK = _NROW // 128
_TAIL_OFF = _TAIL_BLOCK * 128
_TAIL_LEN = _NROW - _TAIL_OFF
_F_SID = 0
_F_NEW = 4
_F_PAR = 8
_F_PBLK = 12
_F_PPAR = 16
_F_PLEN = 20
_F_SCAL = 24
_META_ROWS = 25


def _issue(params_hbm, blk_v, sems, slot, is_tail, block):
    off = pl.multiple_of(block * 128, 128)
    for s in range(_NBUF):
        @pl.when(jnp.logical_and(slot == s, is_tail == 0))
        def _():
            pltpu.async_copy(
                params_hbm.at[:, pl.ds(off, 128)],
                blk_v.at[s],
                sems[s],
            )


def _wait(params_hbm, blk_v, sems, slot):
    for s in range(_NBUF):
        @pl.when(slot == s)
        def _():
            pltpu.make_async_copy(
                params_hbm.at[:, pl.ds(0, 128)],
                blk_v.at[s],
                sems[s],
            ).wait()


def _body(params_hbm, tail_hbm, meta_hbm, out_hbm, meta_v, blk_v, rows_v,
          *sems):
    wid = lax.axis_index("s") * 2 + lax.axis_index("c")
    pltpu.sync_copy(meta_hbm.at[wid], meta_v)
    pltpu.sync_copy(tail_hbm, blk_v.at[_TAIL_SLOT])

    scal = meta_v[_F_SCAL, pl.ds(0, 16)]
    for i in range(_LOOK):
        _issue(params_hbm, blk_v, sems,
               scal[_LOOK + i], scal[2 * _LOOK + i], scal[i])

    iotas = [lax.iota(jnp.int32, 16) + 16 * g for g in range(4)]

    def chunk(k16, _):
        row = k16 >> 3
        col = (k16 & 7) * 16
        sid16 = meta_v[_F_SID + row, pl.ds(col, 16)]
        new16 = meta_v[_F_NEW + row, pl.ds(col, 16)]
        par16 = meta_v[_F_PAR + row, pl.ds(col, 16)]
        pblk16 = meta_v[_F_PBLK + row, pl.ds(col, 16)]
        ppar16 = meta_v[_F_PPAR + row, pl.ds(col, 16)]
        plen16 = meta_v[_F_PLEN + row, pl.ds(col, 16)]
        ir16 = sid16 & 127
        for l in range(16):
            @pl.when(new16[l] != 0)
            def _():
                _issue(params_hbm, blk_v, sems,
                       ppar16[l], plen16[l], pblk16[l])
                _wait(params_hbm, blk_v, sems, par16[l])
            slot_v = jnp.full((16,), par16[l], jnp.int32)
            ir_v = jnp.full((16,), ir16[l], jnp.int32)
            k = k16 * 16 + l
            for g in range(4):
                vals = plsc.load_gather(blk_v, [slot_v, iotas[g], ir_v])
                rows_v[k, pl.ds(16 * g, 16)] = vals
        return 0

    lax.fori_loop(0, _B_PER_W // 16, chunk, 0)

    for i in range(_LOOK):
        _wait(params_hbm, blk_v, sems, scal[3 * _LOOK + i])

    pltpu.sync_copy(rows_v, out_hbm.at[pl.ds(wid * _B_PER_W, _B_PER_W), :])


def _build_meta(sids):
    s2 = sids.reshape(_NW, _B_PER_W)
    blk = s2 >> 7
    first = jnp.ones((_NW, 1), dtype=bool)
    new = jnp.concatenate([first, blk[:, 1:] != blk[:, :-1]], axis=1)
    seg_of_k = jnp.cumsum(new.astype(jnp.int32), axis=1) - 1
    nseg = seg_of_k[:, -1] + 1

    big = jnp.int32(2 * _B_PER_W)
    r = jnp.arange(_B_PER_W, dtype=jnp.int32)[None, :]
    cand = jnp.where(new, r, big)
    sfx = lax.cummin(cand, axis=1, reverse=True)
    nn = jnp.concatenate(
        [sfx[:, 1:], jnp.full((_NW, 1), big, jnp.int32)], axis=1)

    def step(pos):
        posc = jnp.clip(pos, 0, _B_PER_W - 1)
        return jnp.where(pos >= _B_PER_W, big,
                         jnp.take_along_axis(nn, posc, axis=1))

    nnL = nn
    for _ in range(_LOOK - 1):
        nnL = step(nnL)
    validL = nnL < _B_PER_W
    pblk = jnp.where(
        validL,
        jnp.take_along_axis(blk, jnp.clip(nnL, 0, _B_PER_W - 1), axis=1), 0)
    plen = jnp.logical_and(validL, pblk == _TAIL_BLOCK).astype(jnp.int32)
    ppar = jnp.where(plen != 0, _TAIL_SLOT,
                     (seg_of_k + _LOOK) % _NBUF).astype(jnp.int32)
    par = jnp.where(blk == _TAIL_BLOCK, _TAIL_SLOT,
                    seg_of_k % _NBUF).astype(jnp.int32)

    def field(a):
        return a.astype(jnp.int32).reshape(_NW, 4, 128)

    aw = jnp.arange(_NW)
    pos = jnp.zeros((_NW,), jnp.int32)
    pblks, pslots, ptails = [], [], []
    for i in range(_LOOK):
        ok = pos < _B_PER_W
        posc = jnp.clip(pos, 0, _B_PER_W - 1)
        b_i = jnp.where(ok, blk[aw, posc], 0)
        t_i = jnp.logical_and(ok, b_i == _TAIL_BLOCK).astype(jnp.int32)
        pblks.append(b_i)
        ptails.append(t_i)
        pslots.append(jnp.where(t_i != 0, _TAIL_SLOT, i).astype(jnp.int32))
        pos = jnp.where(ok, nn[aw, posc], big)
    d = [((nseg + i) % _NBUF).astype(jnp.int32) for i in range(_LOOK)]
    scal = jnp.zeros((_NW, 1, 128), jnp.int32)
    for i in range(_LOOK):
        scal = scal.at[:, 0, i].set(pblks[i])
        scal = scal.at[:, 0, _LOOK + i].set(pslots[i])
        scal = scal.at[:, 0, 2 * _LOOK + i].set(ptails[i])
        scal = scal.at[:, 0, 3 * _LOOK + i].set(d[i])
    return jnp.concatenate(
        [field(s2), field(new), field(par), field(pblk), field(ppar),
         field(plen), scal], axis=1)


@jax.jit
def kernel(x, params):
    ids = x[0]
    order = jnp.argsort(ids)
    sids = ids[order]
    meta = _build_meta(sids)
    run = pl.kernel(
        _body,
        out_type=jax.ShapeDtypeStruct((_BATCH, _DIM), jnp.float32),
        mesh=plsc.VectorSubcoreMesh(core_axis_name="c", subcore_axis_name="s"),
        scratch_types=[
            pltpu.VMEM((_META_ROWS, 128), jnp.int32),
            pltpu.VMEM((_NBUF + 1, _DIM, 128), jnp.float32),
            pltpu.VMEM((_B_PER_W, _DIM), jnp.float32),
        ] + [pltpu.SemaphoreType.DMA] * _NBUF,
        compiler_params=pltpu.CompilerParams(
            use_tc_tiling_on_sc=True, needs_layout_passes=False),
    )
    p_t = params.T
    tail = jnp.pad(p_t[:, _TAIL_OFF:], ((0, 0), (0, 128 - _TAIL_LEN)))
    out_sorted = run(p_t, tail, meta)
    inv = jnp.argsort(order)
    return out_sorted[inv]


# speedup vs baseline: 1.3388x; 1.3388x over previous
import jax
import jax.numpy as jnp
from jax import lax
from jax.experimental import pallas as pl
from jax.experimental.pallas import tpu as pltpu
from jax.experimental.pallas import tpu_sc as plsc

_NW = 32
_BATCH = 16384
_DIM = 64
_NROW = 1000000
_B_PER_W = _BATCH // _NW
_NSEG_PAD = 528
_TAIL_BLOCK = _NROW // 128
_TAIL_OFF = _TAIL_BLOCK * 128
_TAIL_LEN = _NROW - _TAIL_OFF
_F_SID = 0
_F_NEW = 4
_F_PAR = 8
_F_PBLK = 12
_F_PPAR = 16
_F_PLEN = 20
_F_SCAL = 24
_META_ROWS = 25


def _issue(params_hbm, blk_v, sems, slot, is_tail, block):
    off = pl.multiple_of(block * 128, 128)
    for s in range(3):
        @pl.when(jnp.logical_and(slot == s, is_tail == 0))
        def _():
            pltpu.async_copy(
                params_hbm.at[:, pl.ds(off, 128)],
                blk_v.at[s],
                sems[s],
            )


def _wait(params_hbm, blk_v, sems, slot):
    for s in range(3):
        @pl.when(slot == s)
        def _():
            pltpu.make_async_copy(
                params_hbm.at[:, pl.ds(0, 128)],
                blk_v.at[s],
                sems[s],
            ).wait()


def _body(params_hbm, tail_hbm, meta_hbm, out_hbm, meta_v, blk_v, rows_v,
          sem0, sem1, sem2):
    sems = (sem0, sem1, sem2)
    wid = lax.axis_index("s") * 2 + lax.axis_index("c")
    pltpu.sync_copy(meta_hbm.at[wid], meta_v)
    pltpu.sync_copy(tail_hbm, blk_v.at[3])

    scal = meta_v[_F_SCAL, pl.ds(0, 16)]
    _issue(params_hbm, blk_v, sems, scal[2] * 3, scal[2], scal[0])
    _issue(params_hbm, blk_v, sems,
           jnp.where(scal[3] != 0, 3, 1), scal[3], scal[1])

    iotas = [lax.iota(jnp.int32, 16) + 16 * g for g in range(4)]

    def chunk(k16, _):
        row = k16 >> 3
        col = (k16 & 7) * 16
        sid16 = meta_v[_F_SID + row, pl.ds(col, 16)]
        new16 = meta_v[_F_NEW + row, pl.ds(col, 16)]
        par16 = meta_v[_F_PAR + row, pl.ds(col, 16)]
        pblk16 = meta_v[_F_PBLK + row, pl.ds(col, 16)]
        ppar16 = meta_v[_F_PPAR + row, pl.ds(col, 16)]
        plen16 = meta_v[_F_PLEN + row, pl.ds(col, 16)]
        ir16 = sid16 & 127
        for l in range(16):
            @pl.when(new16[l] != 0)
            def _():
                _issue(params_hbm, blk_v, sems,
                       ppar16[l], plen16[l], pblk16[l])
                _wait(params_hbm, blk_v, sems, par16[l])
            slot_v = jnp.full((16,), par16[l], jnp.int32)
            ir_v = jnp.full((16,), ir16[l], jnp.int32)
            k = k16 * 16 + l
            for g in range(4):
                vals = plsc.load_gather(blk_v, [slot_v, iotas[g], ir_v])
                rows_v[k, pl.ds(16 * g, 16)] = vals
        return 0

    lax.fori_loop(0, _B_PER_W // 16, chunk, 0)

    _wait(params_hbm, blk_v, sems, scal[4])
    _wait(params_hbm, blk_v, sems, scal[5])

    pltpu.sync_copy(rows_v, out_hbm.at[pl.ds(wid * _B_PER_W, _B_PER_W), :])


def _build_meta(sids):
    s2 = sids.reshape(_NW, _B_PER_W)
    blk = s2 >> 7
    first = jnp.ones((_NW, 1), dtype=bool)
    new = jnp.concatenate([first, blk[:, 1:] != blk[:, :-1]], axis=1)
    seg_of_k = jnp.cumsum(new.astype(jnp.int32), axis=1) - 1
    nseg = seg_of_k[:, -1] + 1

    big = jnp.int32(2 * _B_PER_W)
    r = jnp.arange(_B_PER_W, dtype=jnp.int32)[None, :]
    cand = jnp.where(new, r, big)
    sfx = lax.cummin(cand, axis=1, reverse=True)
    nn = jnp.concatenate(
        [sfx[:, 1:], jnp.full((_NW, 1), big, jnp.int32)], axis=1)
    nnc = jnp.clip(nn, 0, _B_PER_W - 1)
    nn2 = jnp.where(nn >= _B_PER_W, big, jnp.take_along_axis(nn, nnc, axis=1))
    nn2c = jnp.clip(nn2, 0, _B_PER_W - 1)
    valid2 = nn2 < _B_PER_W
    pblk = jnp.where(valid2, jnp.take_along_axis(blk, nn2c, axis=1), 0)
    plen = jnp.logical_and(valid2, pblk == _TAIL_BLOCK).astype(jnp.int32)
    ppar = jnp.where(plen != 0, 3, (seg_of_k + 2) % 3).astype(jnp.int32)
    par = jnp.where(blk == _TAIL_BLOCK, 3, seg_of_k % 3).astype(jnp.int32)

    def field(a):
        return a.astype(jnp.int32).reshape(_NW, 4, 128)

    prime0 = blk[:, 0]
    prime0_t = (prime0 == _TAIL_BLOCK).astype(jnp.int32)
    n0 = nn[:, 0]
    has1 = n0 < _B_PER_W
    p1 = jnp.where(has1, blk[jnp.arange(_NW), jnp.clip(n0, 0, _B_PER_W - 1)], 0)
    prime1_t = jnp.logical_and(has1, p1 == _TAIL_BLOCK).astype(jnp.int32)
    d0 = (nseg % 3).astype(jnp.int32)
    d1 = ((nseg + 1) % 3).astype(jnp.int32)
    scal = jnp.zeros((_NW, 1, 128), jnp.int32)
    scal = scal.at[:, 0, 0].set(prime0).at[:, 0, 1].set(p1)
    scal = scal.at[:, 0, 2].set(prime0_t).at[:, 0, 3].set(prime1_t)
    scal = scal.at[:, 0, 4].set(d0).at[:, 0, 5].set(d1)
    return jnp.concatenate(
        [field(s2), field(new), field(par), field(pblk), field(ppar),
         field(plen), scal], axis=1)


@jax.jit
def kernel(x, params):
    ids = x[0]
    order = jnp.argsort(ids)
    sids = ids[order]
    meta = _build_meta(sids)
    run = pl.kernel(
        _body,
        out_type=jax.ShapeDtypeStruct((_BATCH, _DIM), jnp.float32),
        mesh=plsc.VectorSubcoreMesh(core_axis_name="c", subcore_axis_name="s"),
        scratch_types=[
            pltpu.VMEM((_META_ROWS, 128), jnp.int32),
            pltpu.VMEM((4, _DIM, 128), jnp.float32),
            pltpu.VMEM((_B_PER_W, _DIM), jnp.float32),
            pltpu.SemaphoreType.DMA,
            pltpu.SemaphoreType.DMA,
            pltpu.SemaphoreType.DMA,
        ],
        compiler_params=pltpu.CompilerParams(
            use_tc_tiling_on_sc=True, needs_layout_passes=False),
    )
    p_t = params.T
    tail = jnp.pad(p_t[:, _TAIL_OFF:], ((0, 0), (0, 128 - _TAIL_LEN)))
    out_sorted = run(p_t, tail, meta)
    inv = jnp.argsort(order)
    return out_sorted[inv]


# speedup vs baseline: 1.7467x; 1.3047x over previous
import jax
import jax.numpy as jnp
from jax import lax
from jax.experimental import pallas as pl
from jax.experimental.pallas import tpu as pltpu
from jax.experimental.pallas import tpu_sc as plsc

_NW = 32
_BATCH = 16384
_DIM = 64
_NROW = 1000000
_B_PER_W = _BATCH // _NW
_TAIL_BLOCK = _NROW // 128
_TAIL_OFF = _TAIL_BLOCK * 128
_TAIL_LEN = _NROW - _TAIL_OFF
_SEG_PAD = _B_PER_W + 8


def _issue(params_hbm, blk_v, sems, slot, is_tail, block):
    off = pl.multiple_of(block * 128, 128)
    for s in range(3):
        @pl.when(jnp.logical_and(slot == s, jnp.logical_not(is_tail)))
        def _():
            pltpu.async_copy(
                params_hbm.at[:, pl.ds(off, 128)],
                blk_v.at[s],
                sems[s],
            )


def _wait(params_hbm, blk_v, sems, slot):
    for s in range(3):
        @pl.when(slot == s)
        def _():
            pltpu.make_async_copy(
                params_hbm.at[:, pl.ds(0, 128)],
                blk_v.at[s],
                sems[s],
            ).wait()


def _cycle(p):
    return jnp.where(p == 2, 0, p + 1)


def _body(params_hbm, tail_hbm, idx_hbm, out_hbm, idx_v, blk_v, rows_v,
          seg_s, sem0, sem1, sem2):
    sems = (sem0, sem1, sem2)
    wid = lax.axis_index("s") * 2 + lax.axis_index("c")
    pltpu.sync_copy(idx_hbm.at[wid], idx_v)
    pltpu.sync_copy(tail_hbm, blk_v.at[3])

    def prescan(k16, carry):
        prevb, s = carry
        vec = idx_v[k16 >> 3, pl.ds((k16 & 7) * 16, 16)] >> 7
        for l in range(16):
            b = vec[l]
            new = b != prevb
            s = jnp.where(new, s + 1, s)

            @pl.when(new)
            def _(b=b, s=s):
                seg_s[s] = b
            prevb = b
        return prevb, s

    _, s_last = lax.fori_loop(0, _B_PER_W // 16, prescan,
                              (jnp.int32(-1), jnp.int32(-1)))
    nseg = s_last + 1
    seg_s[nseg] = 0
    seg_s[nseg + 1] = 0

    b0 = seg_s[0]
    b1 = seg_s[1]
    _issue(params_hbm, blk_v, sems, jnp.where(b0 == _TAIL_BLOCK, 3, 0),
           b0 == _TAIL_BLOCK, b0)
    _issue(params_hbm, blk_v, sems, jnp.where(b1 == _TAIL_BLOCK, 3, 1),
           b1 == _TAIL_BLOCK, b1)

    iotas = [lax.iota(jnp.int32, 16) + 16 * g for g in range(4)]

    def chunk(k16, carry):
        prevb, s, p = carry
        sid16 = idx_v[k16 >> 3, pl.ds((k16 & 7) * 16, 16)]
        blk16 = sid16 >> 7
        ir16 = sid16 & 127
        for l in range(16):
            b = blk16[l]
            new = b != prevb
            s = jnp.where(new, s + 1, s)
            p = jnp.where(new, _cycle(p), p)
            slot = jnp.where(b == _TAIL_BLOCK, 3, p)

            @pl.when(new)
            def _(s=s, p=p, slot=slot):
                bp = seg_s[s + 2]
                q = _cycle(_cycle(p))
                _issue(params_hbm, blk_v, sems,
                       jnp.where(bp == _TAIL_BLOCK, 3, q),
                       bp == _TAIL_BLOCK, bp)
                _wait(params_hbm, blk_v, sems, slot)
            slot_v = jnp.full((16,), slot, jnp.int32)
            ir_v = jnp.full((16,), ir16[l], jnp.int32)
            k = k16 * 16 + l
            for g in range(4):
                vals = plsc.load_gather(blk_v, [slot_v, iotas[g], ir_v])
                rows_v[k, pl.ds(16 * g, 16)] = vals
            prevb = b
        return prevb, s, p

    lax.fori_loop(0, _B_PER_W // 16, chunk,
                  (jnp.int32(-1), jnp.int32(-1), jnp.int32(-1)))

    _wait(params_hbm, blk_v, sems, lax.rem(nseg, 3))
    _wait(params_hbm, blk_v, sems, lax.rem(nseg + 1, 3))

    pltpu.sync_copy(rows_v, out_hbm.at[pl.ds(wid * _B_PER_W, _B_PER_W), :])


@jax.jit
def kernel(x, params):
    ids = x[0]
    order = jnp.argsort(ids)
    sids = ids[order]
    idx = sids.reshape(_NW, 4, 128)
    run = pl.kernel(
        _body,
        out_type=jax.ShapeDtypeStruct((_BATCH, _DIM), jnp.float32),
        mesh=plsc.VectorSubcoreMesh(core_axis_name="c", subcore_axis_name="s"),
        scratch_types=[
            pltpu.VMEM((4, 128), jnp.int32),
            pltpu.VMEM((4, _DIM, 128), jnp.float32),
            pltpu.VMEM((_B_PER_W, _DIM), jnp.float32),
            pltpu.SMEM((_SEG_PAD,), jnp.int32),
            pltpu.SemaphoreType.DMA,
            pltpu.SemaphoreType.DMA,
            pltpu.SemaphoreType.DMA,
        ],
        compiler_params=pltpu.CompilerParams(
            use_tc_tiling_on_sc=True, needs_layout_passes=False),
    )
    p_t = params.T
    tail = jnp.pad(p_t[:, _TAIL_OFF:], ((0, 0), (0, 128 - _TAIL_LEN)))
    out_sorted = run(p_t, tail, idx)
    inv = jnp.argsort(order)
    return out_sorted[inv]


# speedup vs baseline: 1.8343x; 1.0501x over previous
import jax
import jax.numpy as jnp
from jax import lax
from jax.experimental import pallas as pl
from jax.experimental.pallas import tpu as pltpu
from jax.experimental.pallas import tpu_sc as plsc

_NW = 32
_BATCH = 16384
_DIM = 64
_NROW = 1000000
_B_PER_W = _BATCH // _NW
_TAIL_BLOCK = _NROW // 128
_TAIL_OFF = _TAIL_BLOCK * 128
_TAIL_LEN = _NROW - _TAIL_OFF
_SEG_PAD = _B_PER_W + 8


def _issue(params_hbm, blk_v, sems, slot, is_tail, block):
    off = pl.multiple_of(block * 128, 128)
    for s in range(3):
        @pl.when(jnp.logical_and(slot == s, jnp.logical_not(is_tail)))
        def _():
            pltpu.async_copy(
                params_hbm.at[:, pl.ds(off, 128)],
                blk_v.at[s],
                sems[s],
            )


def _wait(params_hbm, blk_v, sems, slot):
    for s in range(3):
        @pl.when(slot == s)
        def _():
            pltpu.make_async_copy(
                params_hbm.at[:, pl.ds(0, 128)],
                blk_v.at[s],
                sems[s],
            ).wait()


def _cycle(p):
    return jnp.where(p == 2, 0, p + 1)


def _body(params_hbm, tail_hbm, idx_hbm, out_hbm, idx_v, blk_v, rows_v,
          seg_s, sem0, sem1, sem2):
    sems = (sem0, sem1, sem2)
    wid = lax.axis_index("s") * 2 + lax.axis_index("c")
    pltpu.sync_copy(idx_hbm.at[wid], idx_v)
    pltpu.sync_copy(tail_hbm, blk_v.at[3])

    def prescan(k16, carry):
        prevb, s = carry
        vec = idx_v[k16 >> 3, pl.ds((k16 & 7) * 16, 16)] >> 7
        for l in range(16):
            b = vec[l]
            new = b != prevb
            s = jnp.where(new, s + 1, s)

            @pl.when(new)
            def _(b=b, s=s):
                seg_s[s] = b
            prevb = b
        return prevb, s

    _, s_last = lax.fori_loop(0, _B_PER_W // 16, prescan,
                              (jnp.int32(-1), jnp.int32(-1)))
    nseg = s_last + 1
    seg_s[nseg] = 0
    seg_s[nseg + 1] = 0

    b0 = seg_s[0]
    b1 = seg_s[1]
    _issue(params_hbm, blk_v, sems, jnp.where(b0 == _TAIL_BLOCK, 3, 0),
           b0 == _TAIL_BLOCK, b0)
    _issue(params_hbm, blk_v, sems, jnp.where(b1 == _TAIL_BLOCK, 3, 1),
           b1 == _TAIL_BLOCK, b1)

    iotas = [lax.iota(jnp.int32, 16) + 16 * g for g in range(4)]

    def chunk(k16, carry):
        prevb, s, p = carry
        sid16 = idx_v[k16 >> 3, pl.ds((k16 & 7) * 16, 16)]
        blk16 = sid16 >> 7
        ir16 = sid16 & 127
        for l in range(16):
            b = blk16[l]
            new = b != prevb
            s = jnp.where(new, s + 1, s)
            p = jnp.where(new, _cycle(p), p)
            slot = jnp.where(b == _TAIL_BLOCK, 3, p)

            @pl.when(new)
            def _(s=s, p=p, slot=slot):
                bp = seg_s[s + 2]
                q = _cycle(_cycle(p))
                _issue(params_hbm, blk_v, sems,
                       jnp.where(bp == _TAIL_BLOCK, 3, q),
                       bp == _TAIL_BLOCK, bp)
                _wait(params_hbm, blk_v, sems, slot)
            slot_v = jnp.full((16,), slot, jnp.int32)
            ir_v = jnp.full((16,), ir16[l], jnp.int32)
            k = k16 * 16 + l
            for g in range(4):
                vals = plsc.load_gather(blk_v, [slot_v, iotas[g], ir_v])
                rows_v[k, pl.ds(16 * g, 16)] = vals
            prevb = b
        return prevb, s, p

    lax.fori_loop(0, _B_PER_W // 16, chunk,
                  (jnp.int32(-1), jnp.int32(-1), jnp.int32(-1)))

    _wait(params_hbm, blk_v, sems, lax.rem(nseg, 3))
    _wait(params_hbm, blk_v, sems, lax.rem(nseg + 1, 3))

    pltpu.sync_copy(rows_v, out_hbm.at[pl.ds(wid * _B_PER_W, _B_PER_W), :])


@jax.jit
def kernel(x, params):
    ids = x[0]
    iota = jnp.arange(_BATCH, dtype=jnp.int32)
    sids, order = lax.sort((ids, iota), num_keys=1)
    idx = sids.reshape(_NW, 4, 128)
    run = pl.kernel(
        _body,
        out_type=jax.ShapeDtypeStruct((_BATCH, _DIM), jnp.float32),
        mesh=plsc.VectorSubcoreMesh(core_axis_name="c", subcore_axis_name="s"),
        scratch_types=[
            pltpu.VMEM((4, 128), jnp.int32),
            pltpu.VMEM((4, _DIM, 128), jnp.float32),
            pltpu.VMEM((_B_PER_W, _DIM), jnp.float32),
            pltpu.SMEM((_SEG_PAD,), jnp.int32),
            pltpu.SemaphoreType.DMA,
            pltpu.SemaphoreType.DMA,
            pltpu.SemaphoreType.DMA,
        ],
        compiler_params=pltpu.CompilerParams(
            use_tc_tiling_on_sc=True, needs_layout_passes=False),
    )
    p_t = params.T
    tail = jnp.pad(p_t[:, _TAIL_OFF:], ((0, 0), (0, 128 - _TAIL_LEN)))
    out_sorted = run(p_t, tail, idx)
    inv = jnp.zeros((_BATCH,), jnp.int32).at[order].set(iota)
    return out_sorted[inv]
